# Initial kernel scaffold; baseline (speedup 1.0000x reference)
import jax
import jax.numpy as jnp
from jax.experimental import pallas as pl


def kernel(idIndexes, omegaEmb, epoch, childrenLowerEmbedding, childrenHigherEmbedding, parentsEmbL_, parentsEmbH_, parentRange, leavesRatio):
    raise NotImplementedError("write your pallas kernel here")


# speedup vs baseline: 1.0848x; 1.0848x over previous
import functools

import jax
import jax.numpy as jnp
from jax import lax
from jax.experimental import pallas as pl
from jax.experimental.pallas import tpu as pltpu
from jax.experimental.pallas import tpu_sc as plsc

N = 512
D = 32
I_BLK = 64


def _sc_gather(idx, lower, higher):
    info = plsc.get_sparse_core_info()
    num_workers = info.num_cores * info.num_subcores
    b_per_w = N // num_workers

    mesh = plsc.VectorSubcoreMesh(core_axis_name="c", subcore_axis_name="s")

    @functools.partial(
        pl.kernel,
        mesh=mesh,
        compiler_params=pltpu.CompilerParams(use_tc_tiling_on_sc=False),
        out_type=(
            jax.ShapeDtypeStruct((N, D), jnp.float32),
            jax.ShapeDtypeStruct((N, D), jnp.float32),
        ),
        scratch_types=[
            pltpu.VMEM((b_per_w,), jnp.int32),
            pltpu.VMEM((b_per_w, D), jnp.float32),
            pltpu.VMEM((b_per_w, D), jnp.float32),
            pltpu.SemaphoreType.DMA,
            pltpu.SemaphoreType.DMA,
        ],
    )
    def gather_kernel(idx_hbm, lo_hbm, hi_hbm, outlo_hbm, outhi_hbm,
                      idx_v, lo_v, hi_v, sem_lo, sem_hi):
        wid = lax.axis_index("s") * info.num_cores + lax.axis_index("c")
        base = wid * b_per_w
        pltpu.sync_copy(idx_hbm.at[pl.ds(base, b_per_w)], idx_v)
        cp_lo = pltpu.async_copy(lo_hbm.at[idx_v], lo_v, sem_lo)
        cp_hi = pltpu.async_copy(hi_hbm.at[idx_v], hi_v, sem_hi)
        cp_lo.wait()
        cp_hi.wait()
        pltpu.sync_copy(lo_v, outlo_hbm.at[pl.ds(base, b_per_w)])
        pltpu.sync_copy(hi_v, outhi_hbm.at[pl.ds(base, b_per_w)])

    return gather_kernel(idx, lower, higher)


def _tc_losses(lo, hi, lo_t, hi_t, p_lo, p_hi):
    def body(lo_ref, hi_ref, lot_ref, hit_ref, pl_ref, ph_ref, out_ref):
        step = pl.program_id(0)
        lo_i = lo_ref[...]
        hi_i = hi_ref[...]
        p_l = pl_ref[...]
        p_h = ph_ref[...]
        zero = jnp.float32(0.0)
        partial = (jnp.sum(jnp.maximum(p_l - lo_i, zero))
                   + jnp.sum(jnp.maximum(hi_i - p_h, zero))
                   + jnp.sum(jnp.maximum(p_l - hi_i, zero))
                   + jnp.sum(jnp.maximum(lo_i - p_h, zero)))
        partial -= jnp.sum(jnp.maximum(hi_i - lo_i, zero))
        lo_j = lot_ref[...][None, :, :]
        hi_j = hit_ref[...][None, :, :]
        lo_b = lo_i[:, :, None]
        hi_b = hi_i[:, :, None]
        ov = jnp.minimum(hi_b, hi_j) - jnp.maximum(lo_b, lo_j)
        partial += jnp.sum(jnp.maximum(ov, zero))

        @pl.when(step == 0)
        def _():
            out_ref[0, 0] = zero

        out_ref[0, 0] += partial

    return pl.pallas_call(
        body,
        grid=(N // I_BLK,),
        in_specs=[
            pl.BlockSpec((I_BLK, D), lambda i: (i, 0)),
            pl.BlockSpec((I_BLK, D), lambda i: (i, 0)),
            pl.BlockSpec((D, N), lambda i: (0, 0)),
            pl.BlockSpec((D, N), lambda i: (0, 0)),
            pl.BlockSpec((1, D), lambda i: (0, 0)),
            pl.BlockSpec((1, D), lambda i: (0, 0)),
        ],
        out_specs=pl.BlockSpec(memory_space=pltpu.SMEM),
        out_shape=jax.ShapeDtypeStruct((1, 1), jnp.float32),
    )(lo, hi, lo_t, hi_t, p_lo, p_hi)


def kernel(idIndexes, omegaEmb, epoch, childrenLowerEmbedding,
           childrenHigherEmbedding, parentsEmbL_, parentsEmbH_,
           parentRange, leavesRatio):
    idx = idIndexes.astype(jnp.int32)
    lo, hi = _sc_gather(idx, childrenLowerEmbedding, childrenHigherEmbedding)
    loss = _tc_losses(
        lo, hi,
        lo.T, hi.T,
        parentsEmbL_.reshape(1, D), parentsEmbH_.reshape(1, D),
    )
    return jnp.reshape(loss, ())


# speedup vs baseline: 1.4390x; 1.3265x over previous
import functools

import jax
import jax.numpy as jnp
from jax import lax
from jax.experimental import pallas as pl
from jax.experimental.pallas import tpu as pltpu
from jax.experimental.pallas import tpu_sc as plsc

N = 512
D = 32
I_BLK = 64


def _sc_gather(idx, lower, higher):
    info = plsc.get_sparse_core_info()
    num_workers = info.num_cores * info.num_subcores
    b_per_w = N // num_workers

    mesh = plsc.VectorSubcoreMesh(core_axis_name="c", subcore_axis_name="s")

    @functools.partial(
        pl.kernel,
        mesh=mesh,
        out_type=(
            jax.ShapeDtypeStruct((N, D), jnp.float32),
            jax.ShapeDtypeStruct((N, D), jnp.float32),
        ),
        scratch_types=[
            pltpu.VMEM((b_per_w,), jnp.int32),
            pltpu.VMEM((b_per_w, D), jnp.float32),
            pltpu.VMEM((b_per_w, D), jnp.float32),
            pltpu.SemaphoreType.DMA,
            pltpu.SemaphoreType.DMA,
        ],
    )
    def gather_kernel(idx_hbm, lo_hbm, hi_hbm, outlo_hbm, outhi_hbm,
                      idx_v, lo_v, hi_v, sem_lo, sem_hi):
        wid = lax.axis_index("s") * info.num_cores + lax.axis_index("c")
        base = wid * b_per_w
        pltpu.sync_copy(idx_hbm.at[pl.ds(base, b_per_w)], idx_v)
        iv = idx_v[...]
        copies = []
        for j in range(b_per_w):
            v = iv[j]
            copies.append(pltpu.async_copy(
                lo_hbm.at[pl.ds(v, 1)], lo_v.at[pl.ds(j, 1)], sem_lo))
            copies.append(pltpu.async_copy(
                hi_hbm.at[pl.ds(v, 1)], hi_v.at[pl.ds(j, 1)], sem_hi))
        for cp in copies:
            cp.wait()
        pltpu.sync_copy(lo_v, outlo_hbm.at[pl.ds(base, b_per_w)])
        pltpu.sync_copy(hi_v, outhi_hbm.at[pl.ds(base, b_per_w)])

    return gather_kernel(idx, lower, higher)


def _tc_losses(lo, hi, lo_t, hi_t, p_lo, p_hi):
    def body(lo_ref, hi_ref, lot_ref, hit_ref, pl_ref, ph_ref, out_ref):
        step = pl.program_id(0)
        lo_i = lo_ref[...]
        hi_i = hi_ref[...]
        p_l = pl_ref[...]
        p_h = ph_ref[...]
        zero = jnp.float32(0.0)
        partial = (jnp.sum(jnp.maximum(p_l - lo_i, zero))
                   + jnp.sum(jnp.maximum(hi_i - p_h, zero))
                   + jnp.sum(jnp.maximum(p_l - hi_i, zero))
                   + jnp.sum(jnp.maximum(lo_i - p_h, zero)))
        partial -= jnp.sum(jnp.maximum(hi_i - lo_i, zero))
        lo_j = lot_ref[...][None, :, :]
        hi_j = hit_ref[...][None, :, :]
        lo_b = lo_i[:, :, None]
        hi_b = hi_i[:, :, None]
        ov = jnp.minimum(hi_b, hi_j) - jnp.maximum(lo_b, lo_j)
        partial += jnp.sum(jnp.maximum(ov, zero))

        @pl.when(step == 0)
        def _():
            out_ref[0, 0] = zero

        out_ref[0, 0] += partial

    return pl.pallas_call(
        body,
        grid=(N // I_BLK,),
        in_specs=[
            pl.BlockSpec((I_BLK, D), lambda i: (i, 0)),
            pl.BlockSpec((I_BLK, D), lambda i: (i, 0)),
            pl.BlockSpec((D, N), lambda i: (0, 0)),
            pl.BlockSpec((D, N), lambda i: (0, 0)),
            pl.BlockSpec((1, D), lambda i: (0, 0)),
            pl.BlockSpec((1, D), lambda i: (0, 0)),
        ],
        out_specs=pl.BlockSpec(memory_space=pltpu.SMEM),
        out_shape=jax.ShapeDtypeStruct((1, 1), jnp.float32),
    )(lo, hi, lo_t, hi_t, p_lo, p_hi)


def kernel(idIndexes, omegaEmb, epoch, childrenLowerEmbedding,
           childrenHigherEmbedding, parentsEmbL_, parentsEmbH_,
           parentRange, leavesRatio):
    idx = idIndexes.astype(jnp.int32)
    lo, hi = _sc_gather(idx, childrenLowerEmbedding, childrenHigherEmbedding)
    loss = _tc_losses(
        lo, hi,
        lo.T, hi.T,
        parentsEmbL_.reshape(1, D), parentsEmbH_.reshape(1, D),
    )
    return jnp.reshape(loss, ())


# speedup vs baseline: 2.8622x; 1.9890x over previous
import jax
import jax.numpy as jnp
from jax import lax
from jax.experimental import pallas as pl
from jax.experimental.pallas import tpu as pltpu

N = 512
D = 32
BPG = 8
NB = N // BPG
I_BLK = 64


def _fused(idx, lower_t, higher_t, p_lo, p_hi):

    def body(idx_ref, lo_hbm, hi_hbm, pl_ref, ph_ref, out_ref,
             st_lo, st_hi, glo, ghi, sem_lo, sem_hi):
        lane_w = lax.rem(
            lax.broadcasted_iota(jnp.int32, (1, BPG * 128), 1), 128)

        def issue(b):
            buf = b % 2
            cps = []
            for k in range(BPG):
                v = idx_ref[b * BPG + k]
                base = pl.multiple_of((v // 128) * 128, 128)
                cps.append(pltpu.make_async_copy(
                    lo_hbm.at[:, pl.ds(base, 128)],
                    st_lo.at[buf, :, pl.ds(128 * k, 128)], sem_lo))
                cps.append(pltpu.make_async_copy(
                    hi_hbm.at[:, pl.ds(base, 128)],
                    st_hi.at[buf, :, pl.ds(128 * k, 128)], sem_hi))
            for cp in cps:
                cp.start()
            return cps

        pending = {0: issue(0)}
        for b in range(NB):
            if b + 1 < NB:
                pending[b + 1] = issue(b + 1)
            for cp in pending.pop(b):
                cp.wait()
            buf = b % 2
            cvec = jnp.concatenate(
                [jnp.full((1, 128), lax.rem(idx_ref[b * BPG + k], 128),
                          jnp.int32) for k in range(BPG)], axis=1)
            m = lane_w == cvec
            ext_lo = jnp.sum(
                jnp.reshape(jnp.where(m, st_lo[buf], 0.0), (D, BPG, 128)),
                axis=2)
            ext_hi = jnp.sum(
                jnp.reshape(jnp.where(m, st_hi[buf], 0.0), (D, BPG, 128)),
                axis=2)
            glo[b, :, :] = ext_lo
            ghi[b, :, :] = ext_hi

        lo_tj = jnp.concatenate([glo[b] for b in range(NB)], axis=1)
        hi_tj = jnp.concatenate([ghi[b] for b in range(NB)], axis=1)
        lo_all = jnp.transpose(lo_tj)
        hi_all = jnp.transpose(hi_tj)
        p_l = pl_ref[...]
        p_h = ph_ref[...]
        zero = jnp.float32(0.0)
        total = (jnp.sum(jnp.maximum(p_l - lo_all, zero))
                 + jnp.sum(jnp.maximum(hi_all - p_h, zero))
                 + jnp.sum(jnp.maximum(p_l - hi_all, zero))
                 + jnp.sum(jnp.maximum(lo_all - p_h, zero))
                 - jnp.sum(jnp.maximum(hi_all - lo_all, zero)))
        lo_j = lo_tj[None, :, :]
        hi_j = hi_tj[None, :, :]
        for c in range(N // I_BLK):
            lo_b = lo_all[c * I_BLK:(c + 1) * I_BLK, :, None]
            hi_b = hi_all[c * I_BLK:(c + 1) * I_BLK, :, None]
            ov = jnp.minimum(hi_b, hi_j) - jnp.maximum(lo_b, lo_j)
            total += jnp.sum(jnp.maximum(ov, zero))
        out_ref[0, 0] = total

    return pl.pallas_call(
        body,
        grid_spec=pltpu.PrefetchScalarGridSpec(
            num_scalar_prefetch=1,
            grid=(1,),
            in_specs=[
                pl.BlockSpec(memory_space=pl.ANY),
                pl.BlockSpec(memory_space=pl.ANY),
                pl.BlockSpec((1, D), lambda s, i: (0, 0)),
                pl.BlockSpec((1, D), lambda s, i: (0, 0)),
            ],
            out_specs=pl.BlockSpec(memory_space=pltpu.SMEM),
            scratch_shapes=[
                pltpu.VMEM((2, D, BPG * 128), jnp.float32),
                pltpu.VMEM((2, D, BPG * 128), jnp.float32),
                pltpu.VMEM((NB, D, BPG), jnp.float32),
                pltpu.VMEM((NB, D, BPG), jnp.float32),
                pltpu.SemaphoreType.DMA,
                pltpu.SemaphoreType.DMA,
            ],
        ),
        out_shape=jax.ShapeDtypeStruct((1, 1), jnp.float32),
    )(idx, lower_t, higher_t, p_lo, p_hi)


def kernel(idIndexes, omegaEmb, epoch, childrenLowerEmbedding,
           childrenHigherEmbedding, parentsEmbL_, parentsEmbH_,
           parentRange, leavesRatio):
    idx = idIndexes.astype(jnp.int32)
    loss = _fused(
        idx, childrenLowerEmbedding.T, childrenHigherEmbedding.T,
        parentsEmbL_.reshape(1, D), parentsEmbH_.reshape(1, D),
    )
    return jnp.reshape(loss, ())


# speedup vs baseline: 5.0171x; 1.7529x over previous
import jax
import jax.numpy as jnp
from jax import lax
from jax.experimental import pallas as pl
from jax.experimental.pallas import tpu as pltpu

N = 512
D = 32
BPG = 8
NB = N // BPG
NBUF = 6
I_BLK = 64


def _fused(idx, lower_t, higher_t, p_lo, p_hi):

    def body(idx_ref, lo_hbm, hi_hbm, pl_ref, ph_ref, out_ref,
             st_lo, st_hi, glo, ghi, sem_lo, sem_hi):
        lane_w = lax.rem(
            lax.broadcasted_iota(jnp.int32, (1, BPG * 128), 1), 128)

        def issue(b):
            buf = b % NBUF
            cps = []
            for k in range(BPG):
                v = idx_ref[b * BPG + k]
                base = pl.multiple_of((v // 128) * 128, 128)
                cps.append(pltpu.make_async_copy(
                    lo_hbm.at[:, pl.ds(base, 128)],
                    st_lo.at[buf, :, pl.ds(128 * k, 128)], sem_lo))
                cps.append(pltpu.make_async_copy(
                    hi_hbm.at[:, pl.ds(base, 128)],
                    st_hi.at[buf, :, pl.ds(128 * k, 128)], sem_hi))
            for cp in cps:
                cp.start()
            return cps

        pending = {i: issue(i) for i in range(NBUF - 1)}
        for b in range(NB):
            nxt = b + NBUF - 1
            if nxt < NB:
                pending[nxt] = issue(nxt)
            for cp in pending.pop(b):
                cp.wait()
            buf = b % NBUF
            cvec = jnp.concatenate(
                [jnp.full((1, 128), lax.rem(idx_ref[b * BPG + k], 128),
                          jnp.int32) for k in range(BPG)], axis=1)
            m = lane_w == cvec
            ext_lo = jnp.sum(
                jnp.reshape(jnp.where(m, st_lo[buf], 0.0), (D, BPG, 128)),
                axis=2)
            ext_hi = jnp.sum(
                jnp.reshape(jnp.where(m, st_hi[buf], 0.0), (D, BPG, 128)),
                axis=2)
            glo[b, :, :] = ext_lo
            ghi[b, :, :] = ext_hi

        lo_tj = jnp.concatenate([glo[b] for b in range(NB)], axis=1)
        hi_tj = jnp.concatenate([ghi[b] for b in range(NB)], axis=1)
        lo_all = jnp.transpose(lo_tj)
        hi_all = jnp.transpose(hi_tj)
        p_l = pl_ref[...]
        p_h = ph_ref[...]
        zero = jnp.float32(0.0)
        total = (jnp.sum(jnp.maximum(p_l - lo_all, zero))
                 + jnp.sum(jnp.maximum(hi_all - p_h, zero))
                 + jnp.sum(jnp.maximum(p_l - hi_all, zero))
                 + jnp.sum(jnp.maximum(lo_all - p_h, zero))
                 - jnp.sum(jnp.maximum(hi_all - lo_all, zero)))
        lo_j = lo_tj[None, :, :]
        hi_j = hi_tj[None, :, :]
        for c in range(N // I_BLK):
            lo_b = lo_all[c * I_BLK:(c + 1) * I_BLK, :, None]
            hi_b = hi_all[c * I_BLK:(c + 1) * I_BLK, :, None]
            ov = jnp.minimum(hi_b, hi_j) - jnp.maximum(lo_b, lo_j)
            total += jnp.sum(jnp.maximum(ov, zero))
        out_ref[0, 0] = total

    return pl.pallas_call(
        body,
        grid_spec=pltpu.PrefetchScalarGridSpec(
            num_scalar_prefetch=1,
            grid=(1,),
            in_specs=[
                pl.BlockSpec(memory_space=pl.ANY),
                pl.BlockSpec(memory_space=pl.ANY),
                pl.BlockSpec((1, D), lambda s, i: (0, 0)),
                pl.BlockSpec((1, D), lambda s, i: (0, 0)),
            ],
            out_specs=pl.BlockSpec(memory_space=pltpu.SMEM),
            scratch_shapes=[
                pltpu.VMEM((NBUF, D, BPG * 128), jnp.float32),
                pltpu.VMEM((NBUF, D, BPG * 128), jnp.float32),
                pltpu.VMEM((NB, D, BPG), jnp.float32),
                pltpu.VMEM((NB, D, BPG), jnp.float32),
                pltpu.SemaphoreType.DMA,
                pltpu.SemaphoreType.DMA,
            ],
        ),
        out_shape=jax.ShapeDtypeStruct((1, 1), jnp.float32),
    )(idx, lower_t, higher_t, p_lo, p_hi)


def kernel(idIndexes, omegaEmb, epoch, childrenLowerEmbedding,
           childrenHigherEmbedding, parentsEmbL_, parentsEmbH_,
           parentRange, leavesRatio):
    idx = idIndexes.astype(jnp.int32)
    loss = _fused(
        idx, childrenLowerEmbedding.T, childrenHigherEmbedding.T,
        parentsEmbL_.reshape(1, D), parentsEmbH_.reshape(1, D),
    )
    return jnp.reshape(loss, ())
